# baseline (speedup 1.0000x reference)
import functools

import jax
import jax.numpy as jnp
from jax.experimental import pallas as pl
from jax.experimental.pallas import tpu as pltpu

_T = 2048


def _rvq_body(x_ref, cb_ref, out_ref, aux_ref):
    x = x_ref[0]
    t, d = x.shape
    num_q, k, _ = cb_ref.shape
    iota = jax.lax.broadcasted_iota(jnp.int32, (t, k), 1).astype(jnp.float32)
    r = x
    qout = jnp.zeros_like(x)
    aux = jnp.float32(0.0)
    for q in range(num_q):
        cb = cb_ref[q]
        cbsq = jnp.sum(cb * cb, axis=-1)
        dot = jax.lax.dot_general(
            r, cb, (((1,), (1,)), ((), ())),
            preferred_element_type=jnp.float32,
            precision=jax.lax.Precision.DEFAULT,
        )
        rsq = jnp.sum(r * r, axis=-1, keepdims=True)
        dists = rsq - 2.0 * dot + cbsq[None, :]
        m = jnp.min(dists, axis=-1, keepdims=True)
        idx = jnp.min(
            jnp.where(dists <= m, iota, jnp.float32(k)), axis=-1, keepdims=True
        )
        onehot = (iota == idx).astype(jnp.bfloat16)
        cb_hi = cb.astype(jnp.bfloat16)
        cb_lo = (cb - cb_hi.astype(jnp.float32)).astype(jnp.bfloat16)
        cb_cat = jnp.concatenate([cb_hi, cb_lo], axis=1)
        quant_cat = jax.lax.dot_general(
            onehot, cb_cat, (((1,), (0,)), ((), ())),
            preferred_element_type=jnp.float32,
        )
        quant = quant_cat[:, :d] + quant_cat[:, d:]
        qout = qout + quant
        aux = aux + jnp.sum((quant - r) ** 2)
        r = r - quant
    out_ref[...] = qout[None]
    aux_ref[...] = jnp.reshape(aux, (1, 1, 1, 1))


@functools.partial(jax.jit, static_argnames=())
def kernel(faces, face_edges, codebooks):
    del face_edges
    b, n, d = faces.shape
    num_q, k, _ = codebooks.shape
    nb = n // _T
    grid = (b * nb,)
    quant, aux_partials = pl.pallas_call(
        _rvq_body,
        grid=grid,
        in_specs=[
            pl.BlockSpec((1, _T, d), lambda i: (i // nb, i % nb, 0)),
            pl.BlockSpec((num_q, k, d), lambda i: (0, 0, 0)),
        ],
        out_specs=[
            pl.BlockSpec((1, _T, d), lambda i: (i // nb, i % nb, 0)),
            pl.BlockSpec((1, 1, 1, 1), lambda i: (i // nb, i % nb, 0, 0)),
        ],
        out_shape=[
            jax.ShapeDtypeStruct((b, n, d), jnp.float32),
            jax.ShapeDtypeStruct((b, nb, 1, 1), jnp.float32),
        ],
        compiler_params=pltpu.CompilerParams(
            dimension_semantics=("parallel",),
        ),
    )(faces, codebooks)
    aux_loss = jnp.sum(aux_partials) / jnp.float32(b * n * d)
    return quant, aux_loss


# speedup vs baseline: 1.0567x; 1.0567x over previous
import functools

import jax
import jax.numpy as jnp
from jax.experimental import pallas as pl
from jax.experimental.pallas import tpu as pltpu

_T = 2048


def _rvq_body(x_ref, cb_ref, out_ref, aux_ref):
    x = x_ref[...]
    t, d = x.shape
    num_q, k, _ = cb_ref.shape
    iota = jax.lax.broadcasted_iota(jnp.int32, (t, k), 1).astype(jnp.float32)
    r = x
    qout = jnp.zeros_like(x)
    aux = jnp.float32(0.0)
    for q in range(num_q):
        cb = cb_ref[q]
        cbsq = jnp.sum(cb * cb, axis=-1)
        dot = jax.lax.dot_general(
            r, cb, (((1,), (1,)), ((), ())),
            preferred_element_type=jnp.float32,
            precision=jax.lax.Precision.DEFAULT,
        )
        rsq = jnp.sum(r * r, axis=-1, keepdims=True)
        dists = rsq - 2.0 * dot + cbsq[None, :]
        m = jnp.min(dists, axis=-1, keepdims=True)
        idx = jnp.min(
            jnp.where(dists <= m, iota, jnp.float32(k)), axis=-1, keepdims=True
        )
        onehot = (iota == idx).astype(jnp.bfloat16)
        cb_hi = cb.astype(jnp.bfloat16)
        cb_lo = (cb - cb_hi.astype(jnp.float32)).astype(jnp.bfloat16)
        cb_cat = jnp.concatenate([cb_hi, cb_lo], axis=1)
        quant_cat = jax.lax.dot_general(
            onehot, cb_cat, (((1,), (0,)), ((), ())),
            preferred_element_type=jnp.float32,
        )
        quant = quant_cat[:, :d] + quant_cat[:, d:]
        qout = qout + quant
        aux = aux + jnp.sum((quant - r) ** 2)
        r = r - quant
    out_ref[...] = qout
    aux_ref[...] = jnp.reshape(aux, (1, 1, 1))


@functools.partial(jax.jit, static_argnames=())
def kernel(faces, face_edges, codebooks):
    del face_edges
    b, n, d = faces.shape
    num_q, k, _ = codebooks.shape
    tokens = b * n
    flat = faces.reshape(tokens, d)
    grid = (tokens // _T,)
    quant, aux_partials = pl.pallas_call(
        _rvq_body,
        grid=grid,
        in_specs=[
            pl.BlockSpec((_T, d), lambda i: (i, 0)),
            pl.BlockSpec((num_q, k, d), lambda i: (0, 0, 0)),
        ],
        out_specs=[
            pl.BlockSpec((_T, d), lambda i: (i, 0)),
            pl.BlockSpec((1, 1, 1), lambda i: (i, 0, 0)),
        ],
        out_shape=[
            jax.ShapeDtypeStruct((tokens, d), jnp.float32),
            jax.ShapeDtypeStruct((grid[0], 1, 1), jnp.float32),
        ],
        compiler_params=pltpu.CompilerParams(
            dimension_semantics=("parallel",),
        ),
    )(flat, codebooks)
    aux_loss = jnp.sum(aux_partials) / jnp.float32(tokens * d)
    return quant.reshape(b, n, d), aux_loss


# speedup vs baseline: 1.1258x; 1.0654x over previous
import functools

import jax
import jax.numpy as jnp
from jax.experimental import pallas as pl
from jax.experimental.pallas import tpu as pltpu

_T = 2048


def _rvq_body(x_ref, cb_ref, out_ref, aux_ref):
    t, d = x_ref.shape
    h = t // 2
    num_q, k, _ = cb_ref.shape
    iota = jax.lax.broadcasted_iota(jnp.int32, (h, k), 1).astype(jnp.float32)
    aux_total = jnp.float32(0.0)
    for s in range(2):
        aux_total += _rvq_half(
            x_ref, cb_ref, out_ref, iota, s * h, h, d, num_q, k
        )
    aux_ref[...] = jnp.reshape(aux_total, (1, 1, 1))


def _rvq_half(x_ref, cb_ref, out_ref, iota, base, h, d, num_q, k):
    x = x_ref[pl.ds(base, h), :]
    r = x
    qout = jnp.zeros_like(x)
    aux = jnp.float32(0.0)
    for q in range(num_q):
        cb = cb_ref[q]
        cbsq = jnp.sum(cb * cb, axis=-1)
        dot = jax.lax.dot_general(
            r, cb, (((1,), (1,)), ((), ())),
            preferred_element_type=jnp.float32,
            precision=jax.lax.Precision.DEFAULT,
        )
        rsq = jnp.sum(r * r, axis=-1, keepdims=True)
        dists = rsq - 2.0 * dot + cbsq[None, :]
        m = jnp.min(dists, axis=-1, keepdims=True)
        idx = jnp.min(
            jnp.where(dists <= m, iota, jnp.float32(k)), axis=-1, keepdims=True
        )
        onehot = (iota == idx).astype(jnp.bfloat16)
        cb_hi = cb.astype(jnp.bfloat16)
        cb_lo = (cb - cb_hi.astype(jnp.float32)).astype(jnp.bfloat16)
        cb_cat = jnp.concatenate([cb_hi, cb_lo], axis=1)
        quant_cat = jax.lax.dot_general(
            onehot, cb_cat, (((1,), (0,)), ((), ())),
            preferred_element_type=jnp.float32,
        )
        quant = quant_cat[:, :d] + quant_cat[:, d:]
        qout = qout + quant
        aux = aux + jnp.sum((quant - r) ** 2)
        r = r - quant
    out_ref[pl.ds(base, h), :] = qout
    return aux


@functools.partial(jax.jit, static_argnames=())
def kernel(faces, face_edges, codebooks):
    del face_edges
    b, n, d = faces.shape
    num_q, k, _ = codebooks.shape
    tokens = b * n
    flat = faces.reshape(tokens, d)
    grid = (tokens // _T,)
    quant, aux_partials = pl.pallas_call(
        _rvq_body,
        grid=grid,
        in_specs=[
            pl.BlockSpec((_T, d), lambda i: (i, 0)),
            pl.BlockSpec((num_q, k, d), lambda i: (0, 0, 0)),
        ],
        out_specs=[
            pl.BlockSpec((_T, d), lambda i: (i, 0)),
            pl.BlockSpec((1, 1, 1), lambda i: (i, 0, 0)),
        ],
        out_shape=[
            jax.ShapeDtypeStruct((tokens, d), jnp.float32),
            jax.ShapeDtypeStruct((grid[0], 1, 1), jnp.float32),
        ],
        compiler_params=pltpu.CompilerParams(
            dimension_semantics=("parallel",),
        ),
    )(flat, codebooks)
    aux_loss = jnp.sum(aux_partials) / jnp.float32(tokens * d)
    return quant.reshape(b, n, d), aux_loss
